# baseline (speedup 1.0000x reference)
import functools

import jax
import jax.numpy as jnp
from jax import lax
from jax.experimental import pallas as pl
from jax.experimental.pallas import tpu as pltpu
from jax.experimental.pallas import tpu_sc as plsc

B, L = 4096, 20
T = B * L
HIDDEN = 768
INTD = HIDDEN // 3
H2 = HIDDEN // 2
EPS = 1e-6

NC, NS = 2, 16
NW = NC * NS
TPW = T // NW
TBK = 64
NBLK = TPW // TBK
NG = TBK // 16

_f32 = jnp.float32


def _prep_body(emb_tag_ref, emb_int_ref, W_cat_ref, b_cat_ref, g1_ref,
               beta1_ref, W_num_ref, b_num_ref, g2_ref, beta2_ref, g3_ref,
               beta3_ref, tab_ref, aux_ref):
    Tt = jnp.dot(emb_tag_ref[...], W_cat_ref[:INTD, :],
                 preferred_element_type=_f32)
    Ti = jnp.dot(emb_int_ref[...], W_cat_ref[INTD:, :],
                 preferred_element_type=_f32)
    r = lax.broadcasted_iota(jnp.int32, (128, 16), 0)
    c = lax.broadcasted_iota(jnp.int32, (128, 16), 1)
    oh_t = ((r // 10) == c).astype(_f32)
    oh_i = ((r % 10) == c).astype(_f32)
    pre = (jnp.dot(oh_t, Tt, preferred_element_type=_f32)
           + jnp.dot(oh_i, Ti, preferred_element_type=_f32)
           + b_cat_ref[...])
    m = jnp.mean(pre, axis=-1, keepdims=True)
    v = jnp.mean((pre - m) ** 2, axis=-1, keepdims=True)
    C = (pre - m) * lax.rsqrt(v + EPS) * g1_ref[...] + beta1_ref[...]
    g3a = g3_ref[:H2]
    g3b = g3_ref[H2:]
    tab_ref[...] = C * g3a
    Sc = jnp.sum(C, axis=1)
    Qc = jnp.sum(C * C, axis=1)

    w = W_num_ref[0, :]
    wc = w - jnp.mean(w)
    bn = b_num_ref[...]
    bc = bn - jnp.mean(bn)
    g2v = g2_ref[...]
    b2v = beta2_ref[...]
    u = wc * g2v
    q = bc * g2v
    scal_rows = jnp.stack([
        wc * wc * (1.0 / H2), wc * bc * (1.0 / H2), bc * bc * (1.0 / H2),
        u, q, b2v, u * u, u * q, q * q, u * b2v, q * b2v, b2v * b2v,
    ])
    scal = jnp.sum(scal_rows, axis=1)
    z = jnp.zeros((H2,), _f32)
    aux_ref[...] = jnp.stack([
        u * g3b,
        q * g3b,
        b2v * g3b,
        g3a,
        beta3_ref[:H2],
        g3b,
        beta3_ref[H2:],
        jnp.concatenate([Sc, jnp.zeros((H2 - 128,), _f32)]),
        jnp.concatenate([Qc, jnp.zeros((H2 - 128,), _f32)]),
        jnp.concatenate([scal, jnp.zeros((H2 - 12,), _f32)]),
        z, z, z, z, z, z,
    ])


def _rsqrt16(x):
    i = plsc.bitcast(x, jnp.int32)
    y = plsc.bitcast(jnp.int32(0x5F3759DF) - lax.shift_right_logical(i, 1),
                     _f32)
    for _ in range(3):
        y = y * (1.5 - 0.5 * x * y * y)
    return y


def _sc_body(tag_h, inter_h, n_h, tab_h, aux_h, out_h,
             tag_v, int_v, combo_v, n_v, rows_v, outb_v, aux_v, scal_v, sem):
    wid = lax.axis_index("s") * NC + lax.axis_index("c")
    base = wid * TPW
    pltpu.sync_copy(aux_h, aux_v)

    def full(val):
        return jnp.full((16,), val, jnp.int32)

    def spl(k):
        return plsc.load_gather(aux_v, [full(9), full(k)])

    Vw, Cwb, Vb = spl(0), spl(1), spl(2)
    Su, Sq, Sb2 = spl(3), spl(4), spl(5)
    Suu, Suq, Sqq = spl(6), spl(7), spl(8)
    Sub, Sqb, Sbb = spl(9), spl(10), spl(11)
    iota = lax.iota(jnp.int32, 16)
    tok_ids = [iota + g * 16 for g in range(NG)]

    @pl.loop(0, NBLK)
    def _block(b):
        t0 = base + b * TBK
        h1 = pltpu.async_copy(tag_h.at[pl.ds(t0, TBK)], tag_v, sem)
        h2 = pltpu.async_copy(inter_h.at[pl.ds(t0, TBK)], int_v, sem)
        h3 = pltpu.async_copy(n_h.at[pl.ds(t0, TBK)], n_v, sem)
        h1.wait()
        h2.wait()
        h3.wait()
        for g in range(NG):
            sl = pl.ds(g * 16, 16)
            combo_v[sl] = tag_v[sl] * 10 + int_v[sl]
        cp = pltpu.async_copy(tab_h.at[combo_v], rows_v, sem)
        for g in range(NG):
            sl = pl.ds(g * 16, 16)
            c16 = combo_v[sl]
            n16 = n_v[sl]
            sc = plsc.load_gather(aux_v, [full(7), c16])
            qc = plsc.load_gather(aux_v, [full(8), c16])
            rr = _rsqrt16(n16 * n16 * Vw + 2.0 * n16 * Cwb + Vb + EPS)
            sum_num = rr * (n16 * Su + Sq) + Sb2
            ssq = (rr * rr * (n16 * n16 * Suu + 2.0 * n16 * Suq + Sqq)
                   + 2.0 * rr * (n16 * Sub + Sqb) + Sbb)
            mean = (sc + sum_num) * (1.0 / HIDDEN)
            ex2 = (qc + ssq) * (1.0 / HIDDEN)
            s = _rsqrt16(ex2 - mean * mean + EPS)
            scal_v[0, sl] = s * rr * n16
            scal_v[1, sl] = s * rr
            scal_v[2, sl] = s
            scal_v[3, sl] = mean * s

        for c in range(4):
            ar = [aux_v[0, pl.ds(c * 96 + jj * 16, 16)] for jj in range(6)]
            br = [aux_v[1, pl.ds(c * 96 + jj * 16, 16)] for jj in range(6)]
            dr = [aux_v[2, pl.ds(c * 96 + jj * 16, 16)] for jj in range(6)]
            gr = [aux_v[5, pl.ds(c * 96 + jj * 16, 16)] for jj in range(6)]
            er = [aux_v[6, pl.ds(c * 96 + jj * 16, 16)] for jj in range(6)]

            @plsc.parallel_loop(0, TBK, unroll=4)
            def _ntok(k):
                kk = jnp.full((16,), k, jnp.int32)
                asp = plsc.load_gather(scal_v, [full(0), kk])
                bsp = plsc.load_gather(scal_v, [full(1), kk])
                gsp = plsc.load_gather(scal_v, [full(2), kk])
                dsp = plsc.load_gather(scal_v, [full(3), kk])
                kh = k * HIDDEN
                for jj in range(6):
                    sl = pl.ds(kh + H2 + c * 96 + jj * 16, 16)
                    outb_v[sl] = (asp * ar[jj] + bsp * br[jj]
                                  + gsp * dr[jj] - dsp * gr[jj] + er[jj])

        cp.wait()
        for c in range(2):
            g1r = [aux_v[3, pl.ds(c * 192 + jj * 16, 16)] for jj in range(12)]
            e1r = [aux_v[4, pl.ds(c * 192 + jj * 16, 16)] for jj in range(12)]

            @plsc.parallel_loop(0, TBK, unroll=4)
            def _tok(k):
                kk = jnp.full((16,), k, jnp.int32)
                gsp = plsc.load_gather(scal_v, [full(2), kk])
                dsp = plsc.load_gather(scal_v, [full(3), kk])
                kh = k * HIDDEN
                for jj in range(12):
                    sl = pl.ds(c * 192 + jj * 16, 16)
                    outb_v[pl.ds(kh + c * 192 + jj * 16, 16)] = \
                        gsp * rows_v[k, sl] - dsp * g1r[jj] + e1r[jj]

        pltpu.sync_copy(outb_v, out_h.at[pl.ds(t0 * HIDDEN, TBK * HIDDEN)])


def kernel(testTag, interaction, num_feat, emb_tag, emb_int, W_cat, b_cat,
           g1, beta1, W_num, b_num, g2, beta2, g3, beta3):
    tag = testTag.reshape(T)
    inter = interaction.reshape(T)
    n = num_feat.reshape(T)
    emb_tag16 = jnp.zeros((16, INTD), _f32).at[:10].set(emb_tag)
    emb_int16 = jnp.zeros((16, INTD), _f32).at[:10].set(emb_int)

    tab, aux = pl.pallas_call(
        _prep_body,
        out_shape=(jax.ShapeDtypeStruct((128, H2), _f32),
                   jax.ShapeDtypeStruct((16, H2), _f32)),
    )(emb_tag16, emb_int16, W_cat, b_cat, g1, beta1, W_num, b_num, g2,
      beta2, g3, beta3)

    mesh = plsc.VectorSubcoreMesh(core_axis_name="c", subcore_axis_name="s",
                                  num_cores=NC, num_subcores=NS)
    sc_call = pl.kernel(
        _sc_body,
        out_type=jax.ShapeDtypeStruct((T * HIDDEN,), _f32),
        mesh=mesh,
        compiler_params=pltpu.CompilerParams(use_tc_tiling_on_sc=False,
                                             needs_layout_passes=False),
        scratch_types=[
            pltpu.VMEM((TBK,), jnp.int32),
            pltpu.VMEM((TBK,), jnp.int32),
            pltpu.VMEM((TBK,), jnp.int32),
            pltpu.VMEM((TBK,), _f32),
            pltpu.VMEM((TBK, H2), _f32),
            pltpu.VMEM((TBK * HIDDEN,), _f32),
            pltpu.VMEM((16, H2), _f32),
            pltpu.VMEM((4, TBK), _f32),
            pltpu.SemaphoreType.DMA,
        ],
    )
    out = sc_call(tag, inter, n, tab, aux)
    return out.reshape(B, L, HIDDEN)


# speedup vs baseline: 1.9471x; 1.9471x over previous
import jax, jax.numpy as jnp, functools
from jax import lax
from jax.experimental import pallas as pl
from jax.experimental.pallas import tpu as pltpu

B, L = 4096, 20
T = B * L
HIDDEN = 768
INTD = 256
H2 = 384
EPS = 1e-6
BB = 256
TB = BB * L
GRID = B // BB


def _ln_rows(x, g, b):
    m = jnp.mean(x, axis=-1, keepdims=True)
    v = jnp.mean((x - m) ** 2, axis=-1, keepdims=True)
    return (x - m) * jax.lax.rsqrt(v + EPS) * g + b


def _body(tag_ref, inter_ref, num_ref, emb_tag_ref, emb_int_ref, W_cat_ref,
          b_cat_ref, g1_ref, beta1_ref, W_num_ref, b_num_ref, g2_ref,
          beta2_ref, g3_ref, beta3_ref, out_ref, tt_scratch, ti_scratch):
    @pl.when(pl.program_id(0) == 0)
    def _():
        tt_scratch[...] = jnp.dot(emb_tag_ref[...], W_cat_ref[:INTD, :],
                                  preferred_element_type=jnp.float32)
        ti_scratch[...] = jnp.dot(emb_int_ref[...], W_cat_ref[INTD:, :],
                                  preferred_element_type=jnp.float32)

    tag = tag_ref[...]
    inter = inter_ref[...]
    n = num_ref[...]
    ids = jax.lax.broadcasted_iota(jnp.int32, (TB, 16), 1)
    oh_t = (tag[:, None] == ids).astype(jnp.float32)
    oh_i = (inter[:, None] == ids).astype(jnp.float32)
    catp = (jnp.dot(oh_t, tt_scratch[...], preferred_element_type=jnp.float32)
            + jnp.dot(oh_i, ti_scratch[...], preferred_element_type=jnp.float32)
            + b_cat_ref[...])
    cat = _ln_rows(catp, g1_ref[...], beta1_ref[...])
    nump = n[:, None] * W_num_ref[0, :] + b_num_ref[...]
    num = _ln_rows(nump, g2_ref[...], beta2_ref[...])
    x = jnp.concatenate([cat, num], axis=-1)
    y = _ln_rows(x, g3_ref[...], beta3_ref[...])
    out_ref[...] = y.reshape(BB, L, HIDDEN)


def kernel(testTag, interaction, num_feat, emb_tag, emb_int, W_cat, b_cat,
           g1, beta1, W_num, b_num, g2, beta2, g3, beta3):
    tag = testTag.reshape(T)
    inter = interaction.reshape(T)
    n = num_feat.reshape(T)
    emb_tag16 = jnp.zeros((16, INTD), jnp.float32).at[:10].set(emb_tag)
    emb_int16 = jnp.zeros((16, INTD), jnp.float32).at[:10].set(emb_int)
    rep = lambda shape: pl.BlockSpec(shape, lambda i: (0,) * len(shape))
    out = pl.pallas_call(
        _body,
        grid=(GRID,),
        in_specs=[
            pl.BlockSpec((TB,), lambda i: (i,)),
            pl.BlockSpec((TB,), lambda i: (i,)),
            pl.BlockSpec((TB,), lambda i: (i,)),
            rep((16, INTD)), rep((16, INTD)), rep((2 * INTD, H2)),
            rep((H2,)), rep((H2,)), rep((H2,)), rep((1, H2)),
            rep((H2,)), rep((H2,)), rep((H2,)), rep((HIDDEN,)),
            rep((HIDDEN,)),
        ],
        out_specs=pl.BlockSpec((BB, L, HIDDEN), lambda i: (i, 0, 0)),
        out_shape=jax.ShapeDtypeStruct((B, L, HIDDEN), jnp.float32),
        scratch_shapes=[
            pltpu.VMEM((16, H2), jnp.float32),
            pltpu.VMEM((16, H2), jnp.float32),
        ],
    )(tag, inter, n, emb_tag16, emb_int16, W_cat, b_cat, g1, beta1,
      W_num, b_num, g2, beta2, g3, beta3)
    return out


# speedup vs baseline: 2.4576x; 1.2622x over previous
import functools

import jax
import jax.numpy as jnp
from jax import lax
from jax.experimental import pallas as pl
from jax.experimental.pallas import tpu as pltpu
from jax.experimental.pallas import tpu_sc as plsc

B, L = 4096, 20
T = B * L
HIDDEN = 768
INTD = HIDDEN // 3
H2 = HIDDEN // 2
EPS = 1e-6

NC, NS = 2, 16
NW = NC * NS
TPW = T // NW

BB = 256
TB = BB * L
GRID = B // BB

_f32 = jnp.float32


def _prep_body(emb_tag_ref, emb_int_ref, W_cat_ref, b_cat_ref, g1_ref,
               beta1_ref, W_num_ref, b_num_ref, g2_ref, beta2_ref, g3_ref,
               beta3_ref, tab_ref, aux_ref):
    Tt = jnp.dot(emb_tag_ref[...], W_cat_ref[:INTD, :],
                 preferred_element_type=_f32)
    Ti = jnp.dot(emb_int_ref[...], W_cat_ref[INTD:, :],
                 preferred_element_type=_f32)
    r = lax.broadcasted_iota(jnp.int32, (128, 16), 0)
    c = lax.broadcasted_iota(jnp.int32, (128, 16), 1)
    oh_t = ((r // 10) == c).astype(_f32)
    oh_i = ((r % 10) == c).astype(_f32)
    pre = (jnp.dot(oh_t, Tt, preferred_element_type=_f32)
           + jnp.dot(oh_i, Ti, preferred_element_type=_f32)
           + b_cat_ref[...])
    m = jnp.mean(pre, axis=-1, keepdims=True)
    v = jnp.mean((pre - m) ** 2, axis=-1, keepdims=True)
    C = (pre - m) * lax.rsqrt(v + EPS) * g1_ref[...] + beta1_ref[...]
    g3a = g3_ref[:H2]
    g3b = g3_ref[H2:]
    tab_ref[...] = C * g3a
    Sc = jnp.sum(C, axis=1)
    Qc = jnp.sum(C * C, axis=1)

    w = W_num_ref[0, :]
    wc = w - jnp.mean(w)
    bn = b_num_ref[...]
    bc = bn - jnp.mean(bn)
    g2v = g2_ref[...]
    b2v = beta2_ref[...]
    u = wc * g2v
    q = bc * g2v
    scal_rows = jnp.stack([
        wc * wc * (1.0 / H2), wc * bc * (1.0 / H2), bc * bc * (1.0 / H2),
        u, q, b2v, u * u, u * q, q * q, u * b2v, q * b2v, b2v * b2v,
    ])
    scal = jnp.sum(scal_rows, axis=1)
    z = jnp.zeros((H2,), _f32)
    aux_ref[...] = jnp.stack([
        u * g3b,
        q * g3b,
        b2v * g3b,
        g3a,
        beta3_ref[:H2],
        g3b,
        beta3_ref[H2:],
        jnp.concatenate([Sc, jnp.zeros((H2 - 128,), _f32)]),
        jnp.concatenate([Qc, jnp.zeros((H2 - 128,), _f32)]),
        jnp.concatenate([scal, jnp.zeros((H2 - 12,), _f32)]),
        z, z, z, z, z, z,
    ])


def _rsqrt16(x):
    i = plsc.bitcast(x, jnp.int32)
    y = plsc.bitcast(jnp.int32(0x5F3759DF) - lax.shift_right_logical(i, 1),
                     _f32)
    for _ in range(3):
        y = y * (1.5 - 0.5 * x * y * y)
    return y


def _sc_stats(tag_h, inter_h, n_h, aux_h, scal_h,
              tag_v, int_v, n_v, aux_v, scal_v, sem):
    wid = lax.axis_index("s") * NC + lax.axis_index("c")
    base = wid * TPW
    pltpu.sync_copy(aux_h, aux_v)
    h1 = pltpu.async_copy(tag_h.at[pl.ds(base, TPW)], tag_v, sem)
    h2 = pltpu.async_copy(inter_h.at[pl.ds(base, TPW)], int_v, sem)
    h3 = pltpu.async_copy(n_h.at[pl.ds(base, TPW)], n_v, sem)
    h1.wait()
    h2.wait()
    h3.wait()

    def full(val):
        return jnp.full((16,), val, jnp.int32)

    def spl(k):
        return plsc.load_gather(aux_v, [full(9), full(k)])

    Vw, Cwb, Vb = spl(0), spl(1), spl(2)
    Su, Sq, Sb2 = spl(3), spl(4), spl(5)
    Suu, Suq, Sqq = spl(6), spl(7), spl(8)
    Sub, Sqb, Sbb = spl(9), spl(10), spl(11)

    @plsc.parallel_loop(0, TPW, 16, unroll=2)
    def _grp(i):
        sl = pl.ds(i, 16)
        c16 = tag_v[sl] * 10 + int_v[sl]
        n16 = n_v[sl]
        sc = plsc.load_gather(aux_v, [full(7), c16])
        qc = plsc.load_gather(aux_v, [full(8), c16])
        rr = _rsqrt16(n16 * n16 * Vw + 2.0 * n16 * Cwb + Vb + EPS)
        sum_num = rr * (n16 * Su + Sq) + Sb2
        ssq = (rr * rr * (n16 * n16 * Suu + 2.0 * n16 * Suq + Sqq)
               + 2.0 * rr * (n16 * Sub + Sqb) + Sbb)
        mean = (sc + sum_num) * (1.0 / HIDDEN)
        ex2 = (qc + ssq) * (1.0 / HIDDEN)
        s = _rsqrt16(ex2 - mean * mean + EPS)
        scal_v[0, sl] = s * rr * n16
        scal_v[1, sl] = s * rr
        scal_v[2, sl] = s
        scal_v[3, sl] = mean * s

    for i in range(4):
        pltpu.sync_copy(scal_v.at[i], scal_h.at[pl.ds(i * T + base, TPW)])


def _tcc_body(tag_ref, inter_ref, al_ref, be_ref, ga_ref, de_ref,
              tab_ref, aux_ref, out_ref):
    combo = tag_ref[...] * 10 + inter_ref[...]
    ids = lax.broadcasted_iota(jnp.int32, (TB, 128), 1)
    oh = (combo[:, None] == ids).astype(_f32)
    rows = jnp.dot(oh, tab_ref[...],
                   preferred_element_type=_f32)
    al = al_ref[...][:, None]
    be = be_ref[...][:, None]
    ga = ga_ref[...][:, None]
    de = de_ref[...][:, None]
    out_cat = ga * rows - de * aux_ref[3, :] + aux_ref[4, :]
    out_num = (al * aux_ref[0, :] + be * aux_ref[1, :] + ga * aux_ref[2, :]
               - de * aux_ref[5, :] + aux_ref[6, :])
    y = jnp.concatenate([out_cat, out_num], axis=-1)
    out_ref[...] = y.reshape(BB, L, HIDDEN)


def kernel(testTag, interaction, num_feat, emb_tag, emb_int, W_cat, b_cat,
           g1, beta1, W_num, b_num, g2, beta2, g3, beta3):
    tag = testTag.reshape(T)
    inter = interaction.reshape(T)
    n = num_feat.reshape(T)
    emb_tag16 = jnp.zeros((16, INTD), _f32).at[:10].set(emb_tag)
    emb_int16 = jnp.zeros((16, INTD), _f32).at[:10].set(emb_int)

    tab, aux = pl.pallas_call(
        _prep_body,
        out_shape=(jax.ShapeDtypeStruct((128, H2), _f32),
                   jax.ShapeDtypeStruct((16, H2), _f32)),
    )(emb_tag16, emb_int16, W_cat, b_cat, g1, beta1, W_num, b_num, g2,
      beta2, g3, beta3)

    mesh = plsc.VectorSubcoreMesh(core_axis_name="c", subcore_axis_name="s",
                                  num_cores=NC, num_subcores=NS)
    scal = pl.kernel(
        _sc_stats,
        out_type=jax.ShapeDtypeStruct((4 * T,), _f32),
        mesh=mesh,
        compiler_params=pltpu.CompilerParams(use_tc_tiling_on_sc=False,
                                             needs_layout_passes=False),
        scratch_types=[
            pltpu.VMEM((TPW,), jnp.int32),
            pltpu.VMEM((TPW,), jnp.int32),
            pltpu.VMEM((TPW,), _f32),
            pltpu.VMEM((16, H2), _f32),
            pltpu.VMEM((4, TPW), _f32),
            pltpu.SemaphoreType.DMA,
        ],
    )(tag, inter, n, aux)

    rep = lambda shape: pl.BlockSpec(shape, lambda i: (0,) * len(shape))
    out = pl.pallas_call(
        _tcc_body,
        grid=(GRID,),
        in_specs=[
            pl.BlockSpec((TB,), lambda i: (i,)),
            pl.BlockSpec((TB,), lambda i: (i,)),
            pl.BlockSpec((TB,), lambda i: (i,)),
            pl.BlockSpec((TB,), lambda i: (i + GRID,)),
            pl.BlockSpec((TB,), lambda i: (i + 2 * GRID,)),
            pl.BlockSpec((TB,), lambda i: (i + 3 * GRID,)),
            rep((128, H2)),
            rep((16, H2)),
        ],
        out_specs=pl.BlockSpec((BB, L, HIDDEN), lambda i: (i, 0, 0)),
        out_shape=jax.ShapeDtypeStruct((B, L, HIDDEN), _f32),
    )(tag, inter, scal, scal, scal, scal, tab, aux)
    return out
